# baseline (speedup 1.0000x reference)
import jax
import jax.numpy as jnp
from jax.experimental import pallas as pl

NUM_CLASSES = 80
K = 300


def _sig_body(logits_ref, out_ref):
    out_ref[...] = jax.nn.sigmoid(logits_ref[...])


def kernel(pred_logits, pred_boxes, orig_target_sizes):
    B, Q, C = pred_logits.shape
    scores = pl.pallas_call(
        _sig_body,
        out_shape=jax.ShapeDtypeStruct((B, Q, C), jnp.float32),
        grid=(B,),
        in_specs=[pl.BlockSpec((1, Q, C), lambda b: (b, 0, 0))],
        out_specs=pl.BlockSpec((1, Q, C), lambda b: (b, 0, 0)),
    )(pred_logits)
    flat = scores.reshape(B, Q * C)
    top_scores, index = jax.lax.top_k(flat, K)
    labels = index - (index // C) * C
    qidx = index // C
    cx, cy, w, h = jnp.split(pred_boxes, 4, axis=-1)
    bb = jnp.concatenate([cx - 0.5 * w, cy - 0.5 * h, cx + 0.5 * w, cy + 0.5 * h], axis=-1)
    scale = jnp.tile(orig_target_sizes, (1, 2))[:, None, :]
    bb = bb * scale
    boxes = jnp.take_along_axis(bb, jnp.broadcast_to(qidx[:, :, None], (B, K, 4)), axis=1)
    return (labels, boxes, top_scores)


# speedup vs baseline: 9.7057x; 9.7057x over previous
import functools

import jax
import jax.numpy as jnp
from jax import lax
from jax.experimental import pallas as pl
from jax.experimental.pallas import tpu as pltpu
from jax.experimental.pallas import tpu_sc as plsc

BATCH = 128
Q = 1000
C = 80
N = Q * C
K = 300
KPAD = 320
NB1 = 1024
NB2 = 256
CAP1 = 4096
CAP2 = 512
NV = N // 16
ROWS_PER_WORKER = BATCH // 32
INT_MIN = -2147483648


def _mono_key(v):
    b = lax.bitcast_convert_type(v, jnp.int32)
    return b ^ ((b >> 31) & 0x7FFFFFFF)


def _make_sc_call():
    mesh = plsc.VectorSubcoreMesh(core_axis_name="c", subcore_axis_name="s")

    @functools.partial(
        pl.kernel,
        mesh=mesh,
        compiler_params=pltpu.CompilerParams(needs_layout_passes=False),
        out_type=[
            jax.ShapeDtypeStruct((BATCH, KPAD), jnp.int32),
            jax.ShapeDtypeStruct((BATCH, KPAD * 4), jnp.float32),
            jax.ShapeDtypeStruct((BATCH, KPAD), jnp.float32),
        ],
        scratch_types=[
            pltpu.VMEM((N,), jnp.float32),
            pltpu.VMEM((16 * NB1,), jnp.int32),
            pltpu.VMEM((NB1,), jnp.int32),
            pltpu.VMEM((CAP1,), jnp.int32),
            pltpu.VMEM((CAP1,), jnp.int32),
            pltpu.VMEM((16 * NB2,), jnp.int32),
            pltpu.VMEM((CAP2,), jnp.int32),
            pltpu.VMEM((CAP2,), jnp.int32),
            pltpu.VMEM((KPAD,), jnp.int32),
            pltpu.VMEM((Q * 4,), jnp.float32),
            pltpu.VMEM((16,), jnp.float32),
            pltpu.VMEM((KPAD,), jnp.int32),
            pltpu.VMEM((KPAD * 4,), jnp.float32),
            pltpu.VMEM((KPAD,), jnp.float32),
        ],
    )
    def sc_call(logits_hbm, boxes_hbm, scale_hbm,
                lab_hbm, box_hbm, sc_hbm,
                xb, h1, ss1, ck, ci, h2, sk, si, qb, bxv, scv,
                olab, obx, osc):
        wid = lax.axis_index("s") * 2 + lax.axis_index("c")
        iota = lax.iota(jnp.int32, 16)
        ones = jnp.ones((16,), jnp.int32)
        zero16 = jnp.zeros((16,), jnp.int32)
        lanebase1 = iota * NB1
        lanebase2 = iota * NB2
        sentk = jnp.full((16,), INT_MIN, jnp.int32)
        qsh = iota >> 2
        c3 = iota & 3
        c3x2 = c3 ^ 2
        lo_mask = c3 < 2

        def row_body(j, _):
            r = wid * ROWS_PER_WORKER + j
            pltpu.sync_copy(logits_hbm.at[r], xb)
            pltpu.sync_copy(boxes_hbm.at[r], bxv)
            pltpu.sync_copy(scale_hbm.at[r], scv)

            def z1(i, _c):
                h1[pl.ds(i * 16, 16)] = zero16
                return 0
            lax.fori_loop(0, 16 * NB1 // 16, z1, 0)

            def z2(i, _c):
                h2[pl.ds(i * 16, 16)] = zero16
                return 0
            lax.fori_loop(0, 16 * NB2 // 16, z2, 0)

            def p1(i, _c):
                k = _mono_key(xb[pl.ds(i * 16, 16)])
                bkt = (k >> 22) + 512
                plsc.addupdate_scatter(h1, [lanebase1 + bkt], ones)
                return 0
            lax.fori_loop(0, NV, p1, 0)

            def sfx(t, cc):
                carry, cnt = cc
                tv = (NB1 // 16 - 1) - t
                base = tv * 16
                tot = h1[pl.ds(base, 16)]
                for l in range(1, 16):
                    tot = tot + h1[pl.ds(l * NB1 + base, 16)]
                rsuf = lax.rev(plsc.cumsum(lax.rev(tot, (0,))), (0,))
                incl = carry + rsuf
                ss1[pl.ds(base, 16)] = incl - tot
                cnt = cnt + jnp.sum(jnp.where(incl >= K, 1, 0))
                return (carry + jnp.sum(tot), cnt)

            _, cntB = lax.fori_loop(0, NB1 // 16, sfx,
                                    (jnp.int32(0), jnp.int32(0)))
            bb = cntB - 1
            bsplat = jnp.broadcast_to(bb, (16,))
            above_splat = plsc.load_gather(ss1, [bsplat])
            above_s = jnp.max(above_splat)

            def p2(i, off):
                k = _mono_key(xb[pl.ds(i * 16, 16)])
                bkt = (k >> 22) + 512
                m = bkt >= bsplat
                cs = plsc.cumsum(jnp.where(m, 1, 0))
                pos = jnp.maximum(jnp.minimum(off + cs - 1, CAP1 - 1), 0)
                plsc.store_scatter(ck, [pos], k, mask=m)
                plsc.store_scatter(ci, [pos], i * 16 + iota, mask=m)
                return off + plsc.all_reduce_population_count(m)

            offv = lax.fori_loop(0, NV, p2, zero16)
            n_cand = jnp.minimum(jnp.max(offv), CAP1)
            ncv = (n_cand + 15) // 16

            def p3(i, _c):
                k = ck[pl.ds(i * 16, 16)]
                tail = (i * 16 + iota) < n_cand
                inb = (((k >> 22) + 512) == bsplat) & tail
                d2 = (k >> 14) & 0xFF
                plsc.addupdate_scatter(h2, [lanebase2 + d2], ones, mask=inb)
                return 0
            lax.fori_loop(0, ncv, p3, 0)

            def sfx2(t, cc):
                carry, cnt = cc
                tv = (NB2 // 16 - 1) - t
                base = tv * 16
                tot = h2[pl.ds(base, 16)]
                for l in range(1, 16):
                    tot = tot + h2[pl.ds(l * NB2 + base, 16)]
                rsuf = lax.rev(plsc.cumsum(lax.rev(tot, (0,))), (0,))
                incl = carry + rsuf
                cnt = cnt + jnp.sum(jnp.where(above_s + incl >= K, 1, 0))
                return (carry + jnp.sum(tot), cnt)

            _, cntD = lax.fori_loop(0, NB2 // 16, sfx2,
                                    (jnp.int32(0), jnp.int32(0)))
            d2splat = jnp.broadcast_to(cntD - 1, (16,))

            def zs(i, _c):
                sk[pl.ds(i * 16, 16)] = sentk
                si[pl.ds(i * 16, 16)] = zero16
                return 0
            lax.fori_loop(0, CAP2 // 16, zs, 0)

            def p4(i, off):
                k = ck[pl.ds(i * 16, 16)]
                ix = ci[pl.ds(i * 16, 16)]
                tail = (i * 16 + iota) < n_cand
                bkt = (k >> 22) + 512
                d2 = (k >> 14) & 0xFF
                m = ((bkt > bsplat) |
                     ((bkt == bsplat) & (d2 >= d2splat))) & tail
                cs = plsc.cumsum(jnp.where(m, 1, 0))
                pos = jnp.maximum(jnp.minimum(off + cs - 1, CAP2 - 1), 0)
                plsc.store_scatter(sk, [pos], k, mask=m)
                plsc.store_scatter(si, [pos], ix, mask=m)
                return off + plsc.all_reduce_population_count(m)

            lax.fori_loop(0, ncv, p4, zero16)

            for ks in [2 << s for s in range(9)]:
                jj = ks >> 1
                while jj >= 1:
                    if jj >= 16:
                        nb = jj // 16
                        lnb = nb.bit_length() - 1

                        def cross(t, _c, ks=ks, nb=nb, lnb=lnb):
                            v = ((t >> lnb) << (lnb + 1)) + (t & (nb - 1))
                            i1 = v * 16
                            i2 = (v + nb) * 16
                            ak = sk[pl.ds(i1, 16)]
                            bk = sk[pl.ds(i2, 16)]
                            ai = si[pl.ds(i1, 16)]
                            bi = si[pl.ds(i2, 16)]
                            dirn = (i1 & ks) == 0
                            cbe = (ak > bk) | ((ak == bk) & (ai < bi))
                            cond = cbe == dirn
                            sk[pl.ds(i1, 16)] = jnp.where(cond, ak, bk)
                            sk[pl.ds(i2, 16)] = jnp.where(cond, bk, ak)
                            si[pl.ds(i1, 16)] = jnp.where(cond, ai, bi)
                            si[pl.ds(i2, 16)] = jnp.where(cond, bi, ai)
                            return 0

                        lax.fori_loop(0, CAP2 // 32, cross, 0)
                    else:
                        def inner(v, _c, ks=ks, jj=jj):
                            base = v * 16
                            ak = sk[pl.ds(base, 16)]
                            ai = si[pl.ds(base, 16)]
                            pidx = base + (iota ^ jj)
                            bk = plsc.load_gather(sk, [pidx])
                            bi = plsc.load_gather(si, [pidx])
                            dirv = ((base + iota) & ks) == 0
                            keepf = (iota & jj) == 0
                            cbe = (ak > bk) | ((ak == bk) & (ai < bi))
                            cond = (cbe == dirv) == keepf
                            sk[pl.ds(base, 16)] = jnp.where(cond, ak, bk)
                            si[pl.ds(base, 16)] = jnp.where(cond, ai, bi)
                            return 0

                        lax.fori_loop(0, CAP2 // 16, inner, 0)
                    jj >>= 1

            def p5(t, _c):
                k = sk[pl.ds(t * 16, 16)]
                ix = si[pl.ds(t * 16, 16)]
                v = lax.bitcast_convert_type(
                    k ^ ((k >> 31) & 0x7FFFFFFF), jnp.float32)
                sc = 1.0 / (1.0 + jnp.exp(-v))
                q = lax.div(ix, C)
                olab[pl.ds(t * 16, 16)] = ix - q * C
                osc[pl.ds(t * 16, 16)] = sc
                qb[pl.ds(t * 16, 16)] = q
                return 0
            lax.fori_loop(0, KPAD // 16, p5, 0)

            scvv = scv[...]

            def p6(t, _c):
                qv = plsc.load_gather(qb, [t * 4 + qsh])
                g = plsc.load_gather(bxv, [qv * 4 + c3])
                p = plsc.load_gather(bxv, [qv * 4 + c3x2])
                res = jnp.where(lo_mask, g - 0.5 * p, p + 0.5 * g)
                obx[pl.ds(t * 16, 16)] = res * scvv
                return 0
            lax.fori_loop(0, KPAD * 4 // 16, p6, 0)

            pltpu.sync_copy(olab, lab_hbm.at[r])
            pltpu.sync_copy(obx, box_hbm.at[r])
            pltpu.sync_copy(osc, sc_hbm.at[r])
            return 0

        lax.fori_loop(0, ROWS_PER_WORKER, row_body, 0)

    return sc_call


_sc_call = _make_sc_call()


def kernel(pred_logits, pred_boxes, orig_target_sizes):
    logits2d = pred_logits.reshape(BATCH, N)
    boxes2d = pred_boxes.reshape(BATCH, Q * 4)
    scale16 = jnp.tile(orig_target_sizes, (1, 8))
    lab_p, box_p, sc_p = _sc_call(logits2d, boxes2d, scale16)
    labels = lab_p[:, :K]
    boxes = box_p.reshape(BATCH, KPAD, 4)[:, :K]
    scores = sc_p[:, :K]
    return (labels, boxes, scores)


# speedup vs baseline: 30.7292x; 3.1661x over previous
import functools

import jax
import jax.numpy as jnp
from jax import lax
from jax.experimental import pallas as pl
from jax.experimental.pallas import tpu as pltpu
from jax.experimental.pallas import tpu_sc as plsc

BATCH = 128
Q = 1000
C = 80
N = Q * C
K = 300
KPAD = 320
NB1 = 1024
NB2 = 256
CAP1 = 4096
CAP2 = 512
ROWS_PER_WORKER = BATCH // 32
INT_MIN = -2147483648


def _mono_key(v):
    b = lax.bitcast_convert_type(v, jnp.int32)
    return b ^ ((b >> 31) & 0x7FFFFFFF)


def _make_sc_call():
    mesh = plsc.VectorSubcoreMesh(core_axis_name="c", subcore_axis_name="s")

    @functools.partial(
        pl.kernel,
        mesh=mesh,
        compiler_params=pltpu.CompilerParams(needs_layout_passes=False),
        out_type=[
            jax.ShapeDtypeStruct((BATCH, KPAD), jnp.int32),
            jax.ShapeDtypeStruct((BATCH, KPAD * 4), jnp.float32),
            jax.ShapeDtypeStruct((BATCH, KPAD), jnp.float32),
        ],
        scratch_types=[
            pltpu.VMEM((N,), jnp.float32),
            pltpu.VMEM((NB1 * 16,), jnp.int32),
            pltpu.VMEM((CAP1,), jnp.int32),
            pltpu.VMEM((CAP1,), jnp.int32),
            pltpu.VMEM((NB2 * 16,), jnp.int32),
            pltpu.VMEM((CAP2,), jnp.int32),
            pltpu.VMEM((CAP2,), jnp.int32),
            pltpu.VMEM((KPAD,), jnp.int32),
            pltpu.VMEM((Q * 4,), jnp.float32),
            pltpu.VMEM((16,), jnp.float32),
            pltpu.VMEM((KPAD,), jnp.int32),
            pltpu.VMEM((KPAD * 4,), jnp.float32),
            pltpu.VMEM((KPAD,), jnp.float32),
        ],
    )
    def sc_call(logits_hbm, boxes_hbm, scale_hbm,
                lab_hbm, box_hbm, sc_hbm,
                xb, h1, ck, ci, h2, sk, si, qb, bxv, scv,
                olab, obx, osc):
        wid = lax.axis_index("s") * 2 + lax.axis_index("c")
        iota = lax.iota(jnp.int32, 16)
        ones = jnp.ones((16,), jnp.int32)
        zero16 = jnp.zeros((16,), jnp.int32)
        sentk = jnp.full((16,), INT_MIN, jnp.int32)
        qsh = iota >> 2
        c3 = iota & 3
        c3x2 = c3 ^ 2
        lo_mask = c3 < 2
        h1base = 8192 + iota
        h2base = iota

        def row_body(j, _):
            r = wid * ROWS_PER_WORKER + j
            pltpu.sync_copy(logits_hbm.at[r], xb)
            pltpu.sync_copy(boxes_hbm.at[r], bxv)
            pltpu.sync_copy(scale_hbm.at[r], scv)

            @plsc.parallel_loop(0, NB1 * 16, 16, unroll=8)
            def z1(i):
                h1[pl.ds(i, 16)] = zero16

            @plsc.parallel_loop(0, NB2 * 16, 16, unroll=8)
            def z2(i):
                h2[pl.ds(i, 16)] = zero16

            @plsc.parallel_loop(0, N, 16, unroll=4)
            def p1(i):
                k = _mono_key(xb[pl.ds(i, 16)])
                idx = ((k >> 18) & -16) + h1base
                plsc.addupdate_scatter(h1, [idx], ones)

            def sfxg(t, cc):
                carry, cnt, aboveg = cc
                base = (63 - t) * 256
                tot = h1[pl.ds(base, 16)]
                for l in range(1, 16):
                    tot = tot + h1[pl.ds(base + l * 16, 16)]
                incl = carry + jnp.sum(tot)
                hit = incl >= K
                first = hit & (cnt == 0)
                aboveg = jnp.where(first, carry, aboveg)
                return (incl, cnt + jnp.where(hit, 1, 0), aboveg)

            _, cntg, aboveg = lax.fori_loop(
                0, NB1 // 16, sfxg, (jnp.int32(0), jnp.int32(0), jnp.int32(0)))
            gg = cntg - 1

            btot = zero16
            for l in range(16):
                btot = btot + plsc.load_gather(h1, [gg * 256 + iota * 16 + l])
            rsuf = lax.rev(plsc.cumsum(lax.rev(btot, (0,))), (0,))
            inclb = aboveg + rsuf
            cntb = jnp.sum(jnp.where(inclb >= K, 1, 0))
            lsel = cntb - 1
            bb = gg * 16 + lsel
            above_s = jnp.sum(jnp.where(iota == lsel, inclb - btot, 0))
            t1 = (bb - 512) * (1 << 22)
            t1splat = jnp.broadcast_to(t1, (16,))

            @plsc.parallel_loop(0, N, 16, unroll=4, carry=zero16)
            def p2(i, off):
                k = _mono_key(xb[pl.ds(i, 16)])
                m = k >= t1splat
                cs = plsc.cumsum(jnp.where(m, 1, 0))
                pos = jnp.maximum(jnp.minimum(off + cs - 1, CAP1 - 1), 0)
                plsc.store_scatter(ck, [pos], k, mask=m)
                plsc.store_scatter(ci, [pos], i + iota, mask=m)
                return off + plsc.all_reduce_population_count(m)

            n_cand = jnp.minimum(jnp.max(p2), CAP1)
            ncand16 = ((n_cand + 15) // 16) * 16

            t1hi = t1 + (1 << 22)
            t1hisplat = jnp.broadcast_to(t1hi, (16,))

            @plsc.parallel_loop(0, ncand16, 16, unroll=2)
            def p3(i):
                k = ck[pl.ds(i, 16)]
                m = (k >= t1splat) & (k < t1hisplat) & ((i + iota) < n_cand)
                idx = ((k >> 10) & 0xFF0) + h2base
                plsc.addupdate_scatter(h2, [idx], ones, mask=m)

            def sfx2(t, cc):
                carry, cnt, aboveg2 = cc
                base = (15 - t) * 256
                tot = h2[pl.ds(base, 16)]
                for l in range(1, 16):
                    tot = tot + h2[pl.ds(base + l * 16, 16)]
                incl = carry + jnp.sum(tot)
                hit = (above_s + incl) >= K
                first = hit & (cnt == 0)
                aboveg2 = jnp.where(first, carry, aboveg2)
                return (incl, cnt + jnp.where(hit, 1, 0), aboveg2)

            _, cntg2, aboveg2 = lax.fori_loop(
                0, NB2 // 16, sfx2, (jnp.int32(0), jnp.int32(0), jnp.int32(0)))
            gg2 = cntg2 - 1
            btot2 = zero16
            for l in range(16):
                btot2 = btot2 + plsc.load_gather(
                    h2, [gg2 * 256 + iota * 16 + l])
            rsuf2 = lax.rev(plsc.cumsum(lax.rev(btot2, (0,))), (0,))
            inclb2 = above_s + aboveg2 + rsuf2
            cntb2 = jnp.sum(jnp.where(inclb2 >= K, 1, 0))
            dd2 = gg2 * 16 + (cntb2 - 1)
            t2 = t1 + dd2 * (1 << 14)
            t2splat = jnp.broadcast_to(t2, (16,))

            @plsc.parallel_loop(0, CAP2, 16, unroll=4)
            def zs(i):
                sk[pl.ds(i, 16)] = sentk
                si[pl.ds(i, 16)] = zero16

            @plsc.parallel_loop(0, ncand16, 16, unroll=2, carry=zero16)
            def p4(i, off):
                k = ck[pl.ds(i, 16)]
                m = (k >= t2splat) & ((i + iota) < n_cand)
                cs = plsc.cumsum(jnp.where(m, 1, 0))
                pos = jnp.maximum(jnp.minimum(off + cs - 1, CAP2 - 1), 0)
                plsc.store_scatter(sk, [pos], k, mask=m)
                plsc.store_scatter(si, [pos], ci[pl.ds(i, 16)], mask=m)
                return off + plsc.all_reduce_population_count(m)

            _ = p4

            for ks in [2 << s for s in range(9)]:
                jj = ks >> 1
                while jj >= 1:
                    if jj >= 16:
                        nb = jj // 16
                        lnb = nb.bit_length() - 1

                        def cross(t, _c, ks=ks, nb=nb, lnb=lnb):
                            v = ((t >> lnb) << (lnb + 1)) + (t & (nb - 1))
                            i1 = v * 16
                            i2 = (v + nb) * 16
                            ak = sk[pl.ds(i1, 16)]
                            bk = sk[pl.ds(i2, 16)]
                            ai = si[pl.ds(i1, 16)]
                            bi = si[pl.ds(i2, 16)]
                            dirn = (i1 & ks) == 0
                            cbe = (ak > bk) | ((ak == bk) & (ai < bi))
                            cond = cbe == dirn
                            sk[pl.ds(i1, 16)] = jnp.where(cond, ak, bk)
                            sk[pl.ds(i2, 16)] = jnp.where(cond, bk, ak)
                            si[pl.ds(i1, 16)] = jnp.where(cond, ai, bi)
                            si[pl.ds(i2, 16)] = jnp.where(cond, bi, ai)
                            return 0

                        lax.fori_loop(0, CAP2 // 32, cross, 0)
                    else:
                        def inner(v, _c, ks=ks, jj=jj):
                            base = v * 16
                            ak = sk[pl.ds(base, 16)]
                            ai = si[pl.ds(base, 16)]
                            pidx = base + (iota ^ jj)
                            bk = plsc.load_gather(sk, [pidx])
                            bi = plsc.load_gather(si, [pidx])
                            dirv = ((base + iota) & ks) == 0
                            keepf = (iota & jj) == 0
                            cbe = (ak > bk) | ((ak == bk) & (ai < bi))
                            cond = (cbe == dirv) == keepf
                            sk[pl.ds(base, 16)] = jnp.where(cond, ak, bk)
                            si[pl.ds(base, 16)] = jnp.where(cond, ai, bi)
                            return 0

                        lax.fori_loop(0, CAP2 // 16, inner, 0)
                    jj >>= 1

            @plsc.parallel_loop(0, KPAD, 16, unroll=2)
            def p5(t):
                k = sk[pl.ds(t, 16)]
                ix = si[pl.ds(t, 16)]
                v = lax.bitcast_convert_type(
                    k ^ ((k >> 31) & 0x7FFFFFFF), jnp.float32)
                sc = 1.0 / (1.0 + jnp.exp(-v))
                q = lax.div(ix, C)
                olab[pl.ds(t, 16)] = ix - q * C
                osc[pl.ds(t, 16)] = sc
                qb[pl.ds(t, 16)] = q

            scvv = scv[...]

            @plsc.parallel_loop(0, KPAD * 4, 16, unroll=4)
            def p6(t):
                qv = plsc.load_gather(qb, [(t >> 2) + qsh])
                g = plsc.load_gather(bxv, [qv * 4 + c3])
                p = plsc.load_gather(bxv, [qv * 4 + c3x2])
                res = jnp.where(lo_mask, g - 0.5 * p, p + 0.5 * g)
                obx[pl.ds(t, 16)] = res * scvv

            pltpu.sync_copy(olab, lab_hbm.at[r])
            pltpu.sync_copy(obx, box_hbm.at[r])
            pltpu.sync_copy(osc, sc_hbm.at[r])
            return 0

        lax.fori_loop(0, ROWS_PER_WORKER, row_body, 0)

    return sc_call


_sc_call = _make_sc_call()


def kernel(pred_logits, pred_boxes, orig_target_sizes):
    logits2d = pred_logits.reshape(BATCH, N)
    boxes2d = pred_boxes.reshape(BATCH, Q * 4)
    scale16 = jnp.tile(orig_target_sizes, (1, 8))
    lab_p, box_p, sc_p = _sc_call(logits2d, boxes2d, scale16)
    labels = lab_p[:, :K]
    boxes = box_p.reshape(BATCH, KPAD, 4)[:, :K]
    scores = sc_p[:, :K]
    return (labels, boxes, scores)
